# baseline (speedup 1.0000x reference)
import functools

import jax
import jax.numpy as jnp
from jax import lax
from jax.experimental import pallas as pl
from jax.experimental.pallas import tpu as pltpu
from jax.experimental.pallas import tpu_sc as plsc

_VOCAB = 1_000_000
_BATCH, _SEQ, _D = 4096, 200, 32
_DP = 128
_B = _BATCH * _SEQ
_NC, _NS = 2, 16
_NW = _NC * _NS
_BPW = _B // _NW
_ROWS_PER_W = _BATCH // _NW
_CB = 2
_CHUNK = _CB * _SEQ
_NCHUNK = _ROWS_PER_W // _CB

_mesh = plsc.VectorSubcoreMesh(core_axis_name="c", subcore_axis_name="s")


@functools.partial(
    pl.kernel,
    out_type=jax.ShapeDtypeStruct((_BATCH, _SEQ, _D), jnp.float32),
    mesh=_mesh,
    scratch_types=[
        pltpu.VMEM((_CHUNK,), jnp.int32),
        pltpu.VMEM((_CHUNK, _DP), jnp.float32),
        pltpu.VMEM((_CB, _SEQ, _D), jnp.float32),
        pltpu.SemaphoreType.DMA,
    ],
    compiler_params=pltpu.CompilerParams(use_tc_tiling_on_sc=True),
)
def _gather_kernel(idx_hbm, table_hbm, out_hbm, idx_v, rows_v, vbuf, sem):
    wid = lax.axis_index("s") * _NC + lax.axis_index("c")
    base = wid * _BPW
    brow = wid * _ROWS_PER_W

    def body(i, carry):
        off = base + i * _CHUNK
        pltpu.sync_copy(idx_hbm.at[pl.ds(off, _CHUNK)], idx_v)
        pltpu.async_copy(table_hbm.at[idx_v], rows_v, sem).wait()

        def repack(r, c):
            a = r // _SEQ
            s = r % _SEQ
            vbuf[a, s, pl.ds(0, 16)] = rows_v[r, pl.ds(0, 16)]
            vbuf[a, s, pl.ds(16, 16)] = rows_v[r, pl.ds(16, 16)]
            return c

        lax.fori_loop(0, _CHUNK, repack, 0)
        pltpu.sync_copy(vbuf, out_hbm.at[pl.ds(brow + i * _CB, _CB)])
        return carry

    lax.fori_loop(0, _NCHUNK, body, 0)


def kernel(input_ids, embedding):
    flat = input_ids.reshape(_B)
    table128 = jnp.pad(embedding, ((0, 0), (0, _DP - _D)))
    return _gather_kernel(flat, table128)


# speedup vs baseline: 1.2372x; 1.2372x over previous
import functools

import jax
import jax.numpy as jnp
from jax import lax
from jax.experimental import pallas as pl
from jax.experimental.pallas import tpu as pltpu
from jax.experimental.pallas import tpu_sc as plsc

_VOCAB = 1_000_000
_BATCH, _SEQ, _D = 4096, 200, 32
_DP = 128
_B = _BATCH * _SEQ
_NC, _NS = 2, 16
_NW = _NC * _NS
_BPW = _B // _NW
_ROWS_PER_W = _BATCH // _NW
_CHUNK = _SEQ
_NCHUNK = _ROWS_PER_W

_mesh = plsc.VectorSubcoreMesh(core_axis_name="c", subcore_axis_name="s")


@functools.partial(
    pl.kernel,
    out_type=jax.ShapeDtypeStruct((_BATCH, _SEQ, _D), jnp.float32),
    mesh=_mesh,
    scratch_types=[
        pltpu.VMEM((_CHUNK,), jnp.int32),
        pltpu.VMEM((_CHUNK,), jnp.int32),
        pltpu.VMEM((_CHUNK, _DP), jnp.float32),
        pltpu.VMEM((_CHUNK, _DP), jnp.float32),
        pltpu.VMEM((1, _SEQ, _D), jnp.float32),
        pltpu.VMEM((1, _SEQ, _D), jnp.float32),
        pltpu.SemaphoreType.DMA,
        pltpu.SemaphoreType.DMA,
        pltpu.SemaphoreType.DMA,
        pltpu.SemaphoreType.DMA,
    ],
    compiler_params=pltpu.CompilerParams(use_tc_tiling_on_sc=True),
)
def _gather_kernel(idx_hbm, table_hbm, out_hbm,
                   idx0, idx1, rows0, rows1, vb0, vb1,
                   sg0, sg1, ss0, ss1):
    wid = lax.axis_index("s") * _NC + lax.axis_index("c")
    base = wid * _BPW
    brow = wid * _ROWS_PER_W
    idxs, rows, vbs = [idx0, idx1], [rows0, rows1], [vb0, vb1]
    sgs, sss = [sg0, sg1], [ss0, ss1]

    def fetch(i, b):
        pltpu.sync_copy(idx_hbm.at[pl.ds(base + i * _CHUNK, _CHUNK)], idxs[b])
        pltpu.async_copy(table_hbm.at[idxs[b]], rows[b], sgs[b])

    fetch(0, 0)
    fetch(1, 1)

    @pl.loop(0, _NCHUNK, step=2)
    def _(g):
        for b in range(2):
            i = g + b
            pltpu.make_async_copy(table_hbm.at[pl.ds(0, _CHUNK)],
                                  rows[b], sgs[b]).wait()
            @pl.when(i >= 2)
            def _():
                pltpu.make_async_copy(vbs[b], out_hbm.at[pl.ds(brow, 1)],
                                      sss[b]).wait()

            @plsc.parallel_loop(0, _SEQ, unroll=4)
            def _(s):
                vbs[b][0, s, pl.ds(0, 16)] = rows[b][s, pl.ds(0, 16)]
                vbs[b][0, s, pl.ds(16, 16)] = rows[b][s, pl.ds(16, 16)]

            @pl.when(i + 2 < _NCHUNK)
            def _():
                fetch(i + 2, b)

            pltpu.async_copy(vbs[b], out_hbm.at[pl.ds(brow + i, 1)], sss[b])

    for b in range(2):
        pltpu.make_async_copy(vbs[b], out_hbm.at[pl.ds(brow, 1)],
                              sss[b]).wait()


def kernel(input_ids, embedding):
    flat = input_ids.reshape(_B)
    table128 = jnp.pad(embedding, ((0, 0), (0, _DP - _D)))
    return _gather_kernel(flat, table128)


# speedup vs baseline: 1.2374x; 1.0002x over previous
import functools

import jax
import jax.numpy as jnp
from jax import lax
from jax.experimental import pallas as pl
from jax.experimental.pallas import tpu as pltpu
from jax.experimental.pallas import tpu_sc as plsc

_VOCAB = 1_000_000
_BATCH, _SEQ, _D = 4096, 200, 32
_DP = 128
_NC, _NS = 2, 16
_NW = _NC * _NS
_ROWS_PER_W = _BATCH // _NW

_mesh = plsc.VectorSubcoreMesh(core_axis_name="c", subcore_axis_name="s")


@functools.partial(
    pl.kernel,
    out_type=jax.ShapeDtypeStruct((_BATCH, _SEQ, _D), jnp.float32),
    mesh=_mesh,
    scratch_types=[
        pltpu.VMEM((1, _SEQ), jnp.int32),
        pltpu.VMEM((1, _SEQ), jnp.int32),
        pltpu.VMEM((_SEQ, _DP), jnp.float32),
        pltpu.VMEM((_SEQ, _DP), jnp.float32),
        pltpu.VMEM((1, _SEQ, _D), jnp.float32),
        pltpu.VMEM((1, _SEQ, _D), jnp.float32),
        pltpu.SemaphoreType.DMA,
        pltpu.SemaphoreType.DMA,
        pltpu.SemaphoreType.DMA,
        pltpu.SemaphoreType.DMA,
    ],
    compiler_params=pltpu.CompilerParams(use_tc_tiling_on_sc=True),
)
def _gather_kernel(idx_hbm, table_hbm, out_hbm,
                   idx0, idx1, rows0, rows1, vb0, vb1,
                   sg0, sg1, ss0, ss1):
    wid = lax.axis_index("s") * _NC + lax.axis_index("c")
    brow = wid * _ROWS_PER_W
    idxs, rows, vbs = [idx0, idx1], [rows0, rows1], [vb0, vb1]
    sgs, sss = [sg0, sg1], [ss0, ss1]

    def fetch(i, b):
        pltpu.sync_copy(idx_hbm.at[pl.ds(brow + i, 1)], idxs[b])
        pltpu.async_copy(table_hbm.at[idxs[b].at[0]], rows[b], sgs[b])

    fetch(0, 0)
    fetch(1, 1)

    @pl.loop(0, _ROWS_PER_W, step=2)
    def _(g):
        for b in range(2):
            i = g + b
            pltpu.make_async_copy(table_hbm.at[idxs[b].at[0]],
                                  rows[b], sgs[b]).wait()
            @pl.when(i >= 2)
            def _():
                pltpu.make_async_copy(vbs[b], out_hbm.at[pl.ds(brow, 1)],
                                      sss[b]).wait()

            @plsc.parallel_loop(0, _SEQ, unroll=4)
            def _(s):
                vbs[b][0, s, pl.ds(0, 16)] = rows[b][s, pl.ds(0, 16)]
                vbs[b][0, s, pl.ds(16, 16)] = rows[b][s, pl.ds(16, 16)]

            @pl.when(i + 2 < _ROWS_PER_W)
            def _():
                fetch(i + 2, b)

            pltpu.async_copy(vbs[b], out_hbm.at[pl.ds(brow + i, 1)], sss[b])

    for b in range(2):
        pltpu.make_async_copy(vbs[b], out_hbm.at[pl.ds(brow, 1)],
                              sss[b]).wait()


def kernel(input_ids, embedding):
    table128 = jnp.pad(embedding, ((0, 0), (0, _DP - _D)))
    return _gather_kernel(input_ids, table128)
